# baseline (speedup 1.0000x reference)
import functools

import jax
import jax.numpy as jnp
from jax.experimental import pallas as pl
from jax.experimental.pallas import tpu as pltpu


def _cheb_block(adj_ref, x_ref, w_ref, b_ref, o_ref, *, blk):
    i = pl.program_id(0)
    acc = jnp.dot(
        adj_ref[...],
        x_ref[...],
        preferred_element_type=jnp.float32,
        precision=jax.lax.Precision.DEFAULT,
    )
    t = x_ref[pl.ds(i * blk, blk), :] + acc
    o_ref[...] = (
        jnp.dot(
            t,
            w_ref[...],
            preferred_element_type=jnp.float32,
            precision=jax.lax.Precision.DEFAULT,
        )
        + b_ref[...]
    )


@jax.jit
def kernel(x, adj, W0, bias):
    n, d_in = x.shape
    d_out = W0.shape[1]
    blk = 200
    assert n % blk == 0
    b2d = bias.reshape(1, d_out)
    return pl.pallas_call(
        functools.partial(_cheb_block, blk=blk),
        grid=(n // blk,),
        in_specs=[
            pl.BlockSpec((blk, n), lambda i: (i, 0)),
            pl.BlockSpec((n, d_in), lambda i: (0, 0)),
            pl.BlockSpec((d_in, d_out), lambda i: (0, 0)),
            pl.BlockSpec((1, d_out), lambda i: (0, 0)),
        ],
        out_specs=pl.BlockSpec((blk, d_out), lambda i: (i, 0)),
        out_shape=jax.ShapeDtypeStruct((n, d_out), x.dtype),
        compiler_params=pltpu.CompilerParams(
            dimension_semantics=("arbitrary",),
        ),
    )(adj, x, W0, b2d)


# speedup vs baseline: 1.0295x; 1.0295x over previous
import functools

import jax
import jax.numpy as jnp
from jax.experimental import pallas as pl
from jax.experimental.pallas import tpu as pltpu


def _cheb_block(adj_ref, x_ref, w_ref, b_ref, o_ref, *, blk):
    i = pl.program_id(0)
    acc = jnp.dot(
        adj_ref[...],
        x_ref[...],
        preferred_element_type=jnp.float32,
        precision=jax.lax.Precision.DEFAULT,
    )
    t = x_ref[pl.ds(i * blk, blk), :] + acc
    o_ref[...] = (
        jnp.dot(
            t,
            w_ref[...],
            preferred_element_type=jnp.float32,
            precision=jax.lax.Precision.DEFAULT,
        )
        + b_ref[...]
    )


@jax.jit
def kernel(x, adj, W0, bias):
    n, d_in = x.shape
    d_out = W0.shape[1]
    blk = 400
    assert n % blk == 0
    b2d = bias.reshape(1, d_out)
    return pl.pallas_call(
        functools.partial(_cheb_block, blk=blk),
        grid=(n // blk,),
        in_specs=[
            pl.BlockSpec((blk, n), lambda i: (i, 0)),
            pl.BlockSpec((n, d_in), lambda i: (0, 0)),
            pl.BlockSpec((d_in, d_out), lambda i: (0, 0)),
            pl.BlockSpec((1, d_out), lambda i: (0, 0)),
        ],
        out_specs=pl.BlockSpec((blk, d_out), lambda i: (i, 0)),
        out_shape=jax.ShapeDtypeStruct((n, d_out), x.dtype),
        compiler_params=pltpu.CompilerParams(
            dimension_semantics=("arbitrary",),
        ),
    )(adj, x, W0, b2d)
